# baseline (speedup 1.0000x reference)
import functools
import jax
import jax.numpy as jnp
from jax import lax
from jax.experimental import pallas as pl
from jax.experimental.pallas import tpu as pltpu
from jax.experimental.pallas import tpu_sc as plsc

N, S, D = 4, 4096, 1024
NW = 32
ROWS_PER_W = (N * S) // NW
C = 16
CHUNKS = ROWS_PER_W // C

_mesh = plsc.VectorSubcoreMesh(core_axis_name="c", subcore_axis_name="s")


@functools.partial(
    pl.kernel,
    mesh=_mesh,
    out_type=jax.ShapeDtypeStruct((N * S, D), jnp.float32),
    scratch_types=[
        pltpu.VMEM((2, C, D), jnp.float32),
        pltpu.VMEM((2, C, D), jnp.float32),
        pltpu.VMEM((2, C, D), jnp.float32),
        pltpu.SemaphoreType.DMA((2,)),
        pltpu.SemaphoreType.DMA((2,)),
        pltpu.SemaphoreType.DMA((2,)),
    ],
    compiler_params=pltpu.CompilerParams(use_tc_tiling_on_sc=True),
)
def _sc_add(x_hbm, enc_hbm, out_hbm, xv, pv, ov, sx, sp, so):
    wid = lax.axis_index("s") * 2 + lax.axis_index("c")
    xrow = wid * ROWS_PER_W
    prow = (wid % 8) * ROWS_PER_W

    def start_in(g, b):
        pltpu.async_copy(
            x_hbm.at[pl.ds(xrow + g * C, C), :], xv.at[b], sx.at[b])
        pltpu.async_copy(
            enc_hbm.at[pl.ds(prow + g * C, C), :], pv.at[b], sp.at[b])

    def wait_in(g, b):
        pltpu.make_async_copy(
            x_hbm.at[pl.ds(xrow + g * C, C), :], xv.at[b], sx.at[b]).wait()
        pltpu.make_async_copy(
            enc_hbm.at[pl.ds(prow + g * C, C), :], pv.at[b], sp.at[b]).wait()

    def start_out(g, b):
        pltpu.async_copy(
            ov.at[b], out_hbm.at[pl.ds(xrow + g * C, C), :], so.at[b])

    def wait_out(g, b):
        pltpu.make_async_copy(
            ov.at[b], out_hbm.at[pl.ds(xrow + g * C, C), :], so.at[b]).wait()

    start_in(0, 0)
    start_in(1, 1)

    def step(g2, carry):
        for b in range(2):
            g = g2 * 2 + b
            wait_in(g, b)

            @pl.when(g >= 2)
            def _drain():
                wait_out(g - 2, b)

            def body(r, c2):
                for j in range(D // 16):
                    s = j * 16
                    ov[b, r, pl.ds(s, 16)] = (
                        xv[b, r, pl.ds(s, 16)] + pv[b, r, pl.ds(s, 16)])
                return c2

            lax.fori_loop(0, C, body, 0)
            start_out(g, b)

            @pl.when(g + 2 < CHUNKS)
            def _prefetch():
                start_in(g + 2, b)
        return carry

    lax.fori_loop(0, CHUNKS // 2, step, 0)
    wait_out(CHUNKS - 2, 0)
    wait_out(CHUNKS - 1, 1)


def kernel(x, encoding):
    out = _sc_add(x.reshape(N * S, D), encoding)
    return out.reshape(x.shape)
